# baseline (speedup 1.0000x reference)
import functools

import jax
import jax.numpy as jnp
from jax import lax
from jax.experimental import pallas as pl
from jax.experimental.pallas import tpu as pltpu
import jax.experimental.pallas.tpu_sc as plsc

N = 50000
E = 800000
G = 512
EPS = 1e-5

NPAD = 51200
EPAD = 819200
GP = 640
NC = 2
NS = 16
NT = NPAD // NS
ET = EPAD // NS
CHE1 = 512
NCH1 = ET // CHE1
CHE2 = 200
NB2 = 4
NCH2 = ET // CHE2
CHN = 640
NPCH = NT // CHN
RB = 2048
NBLK = NPAD // RB


def _mesh():
    return plsc.VectorSubcoreMesh(
        core_axis_name="c", subcore_axis_name="s", num_cores=NC, num_subcores=NS)


def _sc1_body(sid, cid, pid, srcr, dstr, batr, semb, cemb, pemb, z16, z8, z1,
              ones_h,
              xs, xc, xp, aggs, aggc, aggp, cnt, gcnt,
              acc16, acc8, cnt_sp, gcnt_sp,
              ei0, ei1, ed0, ed1, er16_0, er16_1, er8_0, er8_1,
              pi_v, pd_v, pr16, pr8, ones_e, ones_p,
              sem0, sem1, semp):
    c = lax.axis_index("c")
    s = lax.axis_index("s")
    nb = s * NT
    eb = s * ET
    ei = (ei0, ei1)
    ed = (ed0, ed1)
    er16 = (er16_0, er16_1)
    er8 = (er8_0, er8_1)
    sems = (sem0, sem1)
    pltpu.sync_copy(ones_h.at[pl.ds(0, CHE1)], ones_e)
    pltpu.sync_copy(ones_h, ones_p)

    pltpu.sync_copy(z16, acc16.at[pl.ds(nb, NT)])

    @pl.when(c == 0)
    def _():
        pltpu.sync_copy(z8, acc8.at[pl.ds(nb, NT)])

    @pl.when(c == 1)
    def _():
        pltpu.sync_copy(z1, cnt_sp.at[pl.ds(nb, NT)])

        @pl.when(s == 0)
        def _():
            pltpu.sync_copy(z1.at[pl.ds(0, GP)], gcnt_sp)

    plsc.subcore_barrier()

    @pl.when(c == 0)
    def _():
        @pl.loop(0, NPCH)
        def _(k):
            o = nb + k * CHN
            pltpu.sync_copy(sid.at[pl.ds(o, CHN)], pi_v)
            pltpu.sync_copy(pid.at[pl.ds(o, CHN)], pd_v)
            pltpu.async_copy(semb.at[pi_v], pr16, semp)
            pltpu.async_copy(pemb.at[pd_v], pr8, semp)
            pltpu.make_async_copy(xs.at[pl.ds(0, CHN)], pr16, semp).wait()
            pltpu.make_async_copy(xp.at[pl.ds(0, CHN)], pr8, semp).wait()
            pltpu.sync_copy(pr16, xs.at[pl.ds(o, CHN)])
            pltpu.sync_copy(pr8, xp.at[pl.ds(o, CHN)])

    @pl.when(c == 1)
    def _():
        @pl.loop(0, NPCH)
        def _(k):
            o = nb + k * CHN
            pltpu.sync_copy(cid.at[pl.ds(o, CHN)], pi_v)
            pltpu.sync_copy(batr.at[pl.ds(o, CHN)], pd_v)
            pltpu.async_copy(cemb.at[pi_v], pr16, semp).wait()
            pltpu.sync_copy(pr16, xc.at[pl.ds(o, CHN)])
            pltpu.sync_copy(ones_p, gcnt_sp.at[pd_v], add=True)

    plsc.subcore_barrier()

    @pl.when(c == 0)
    def _():
        for b in range(2):
            pltpu.sync_copy(srcr.at[pl.ds(eb + b * CHE1, CHE1)], ei[b])
            pltpu.sync_copy(dstr.at[pl.ds(eb + b * CHE1, CHE1)], ed[b])
            pltpu.async_copy(xs.at[ei[b]], er16[b], sems[b])
            pltpu.async_copy(xp.at[ei[b]], er8[b], sems[b])

        @pl.loop(0, NCH1, step=2)
        def _(k):
            for b in range(2):
                pltpu.make_async_copy(
                    xs.at[pl.ds(0, CHE1)], er16[b], sems[b]).wait()
                pltpu.make_async_copy(
                    xp.at[pl.ds(0, CHE1)], er8[b], sems[b]).wait()
                pltpu.sync_copy(er16[b], acc16.at[ed[b]], add=True)
                pltpu.sync_copy(er8[b], acc8.at[ed[b]], add=True)

                @pl.when(k + (b + 2) < NCH1)
                def _():
                    o = eb + (k + (b + 2)) * CHE1
                    pltpu.sync_copy(srcr.at[pl.ds(o, CHE1)], ei[b])
                    pltpu.sync_copy(dstr.at[pl.ds(o, CHE1)], ed[b])
                    pltpu.async_copy(xs.at[ei[b]], er16[b], sems[b])
                    pltpu.async_copy(xp.at[ei[b]], er8[b], sems[b])

    @pl.when(c == 1)
    def _():
        for b in range(2):
            pltpu.sync_copy(srcr.at[pl.ds(eb + b * CHE1, CHE1)], ei[b])
            pltpu.sync_copy(dstr.at[pl.ds(eb + b * CHE1, CHE1)], ed[b])
            pltpu.async_copy(xc.at[ei[b]], er16[b], sems[b])

        @pl.loop(0, NCH1, step=2)
        def _(k):
            for b in range(2):
                pltpu.make_async_copy(
                    xc.at[pl.ds(0, CHE1)], er16[b], sems[b]).wait()
                pltpu.sync_copy(er16[b], acc16.at[ed[b]], add=True)
                pltpu.sync_copy(ones_e, cnt_sp.at[ed[b]], add=True)

                @pl.when(k + (b + 2) < NCH1)
                def _():
                    o = eb + (k + (b + 2)) * CHE1
                    pltpu.sync_copy(srcr.at[pl.ds(o, CHE1)], ei[b])
                    pltpu.sync_copy(dstr.at[pl.ds(o, CHE1)], ed[b])
                    pltpu.async_copy(xc.at[ei[b]], er16[b], sems[b])

    plsc.subcore_barrier()

    @pl.when(c == 0)
    def _():
        pltpu.sync_copy(acc16.at[pl.ds(nb, NT)], aggs.at[pl.ds(nb, NT)])
        pltpu.sync_copy(acc8.at[pl.ds(nb, NT)], aggp.at[pl.ds(nb, NT)])

    @pl.when(c == 1)
    def _():
        pltpu.sync_copy(acc16.at[pl.ds(nb, NT)], aggc.at[pl.ds(nb, NT)])
        pltpu.sync_copy(cnt_sp.at[pl.ds(nb, NT)], cnt.at[pl.ds(nb, NT)])

        @pl.when(s == 0)
        def _():
            pltpu.sync_copy(gcnt_sp, gcnt)


def _sc1(sid, cid, pid, srcr, dstr, batr, semb, cemb, pemb, z16, z8, z1,
         ones_h):
    f32 = jnp.float32
    i32 = jnp.int32
    out_type = [
        jax.ShapeDtypeStruct((NPAD, 16), f32),
        jax.ShapeDtypeStruct((NPAD, 16), f32),
        jax.ShapeDtypeStruct((NPAD, 8), f32),
        jax.ShapeDtypeStruct((NPAD, 16), f32),
        jax.ShapeDtypeStruct((NPAD, 16), f32),
        jax.ShapeDtypeStruct((NPAD, 8), f32),
        jax.ShapeDtypeStruct((NPAD,), f32),
        jax.ShapeDtypeStruct((GP,), f32),
    ]
    scratch = [
        pltpu.VMEM_SHARED((NPAD, 16), f32),
        pltpu.VMEM_SHARED((NPAD, 8), f32),
        pltpu.VMEM_SHARED((NPAD,), f32),
        pltpu.VMEM_SHARED((GP,), f32),
        pltpu.VMEM((CHE1,), i32),
        pltpu.VMEM((CHE1,), i32),
        pltpu.VMEM((CHE1,), i32),
        pltpu.VMEM((CHE1,), i32),
        pltpu.VMEM((CHE1, 16), f32),
        pltpu.VMEM((CHE1, 16), f32),
        pltpu.VMEM((CHE1, 8), f32),
        pltpu.VMEM((CHE1, 8), f32),
        pltpu.VMEM((CHN,), i32),
        pltpu.VMEM((CHN,), i32),
        pltpu.VMEM((CHN, 16), f32),
        pltpu.VMEM((CHN, 8), f32),
        pltpu.VMEM((CHE1,), f32),
        pltpu.VMEM((CHN,), f32),
        pltpu.SemaphoreType.DMA,
        pltpu.SemaphoreType.DMA,
        pltpu.SemaphoreType.DMA,
    ]
    return pl.kernel(_sc1_body, out_type=out_type, mesh=_mesh(),
                     scratch_types=scratch,
                     compiler_params=pltpu.CompilerParams(
                         use_tc_tiling_on_sc=False))(
        sid, cid, pid, srcr, dstr, batr, semb, cemb, pemb, z16, z8, z1,
        ones_h)


def _sc2_body(srcr, dstr, y1a, y1b, z32, agg2a, agg2b,
              acc32, ei0, ei1, ei2, ei3, ed0, ed1, ed2, ed3,
              er32_0, er32_1, er32_2, er32_3, sem0, sem1, sem2, sem3):
    c = lax.axis_index("c")
    s = lax.axis_index("s")
    nb = s * NT
    eb = s * ET
    ei = (ei0, ei1, ei2, ei3)
    ed = (ed0, ed1, ed2, ed3)
    er32 = (er32_0, er32_1, er32_2, er32_3)
    sems = (sem0, sem1, sem2, sem3)

    pltpu.sync_copy(z32, acc32.at[pl.ds(nb, NT)])
    plsc.subcore_barrier()

    def edge_loop(table):
        for b in range(NB2):
            pltpu.sync_copy(srcr.at[pl.ds(eb + b * CHE2, CHE2)], ei[b])
            pltpu.sync_copy(dstr.at[pl.ds(eb + b * CHE2, CHE2)], ed[b])
            pltpu.async_copy(table.at[ei[b]], er32[b], sems[b])

        @pl.loop(0, NCH2, step=NB2)
        def _(k):
            for b in range(NB2):
                pltpu.make_async_copy(
                    table.at[pl.ds(0, CHE2)], er32[b], sems[b]).wait()
                pltpu.sync_copy(er32[b], acc32.at[ed[b]], add=True)

                @pl.when(k + (b + NB2) < NCH2)
                def _():
                    o = eb + (k + (b + NB2)) * CHE2
                    pltpu.sync_copy(srcr.at[pl.ds(o, CHE2)], ei[b])
                    pltpu.sync_copy(dstr.at[pl.ds(o, CHE2)], ed[b])
                    pltpu.async_copy(table.at[ei[b]], er32[b], sems[b])

    @pl.when(c == 0)
    def _():
        edge_loop(y1a)

    @pl.when(c == 1)
    def _():
        edge_loop(y1b)

    plsc.subcore_barrier()

    @pl.when(c == 0)
    def _():
        pltpu.sync_copy(acc32.at[pl.ds(nb, NT)], agg2a.at[pl.ds(nb, NT)])

    @pl.when(c == 1)
    def _():
        pltpu.sync_copy(acc32.at[pl.ds(nb, NT)], agg2b.at[pl.ds(nb, NT)])


def _sc2(srcr, dstr, y1a, y1b, z32):
    f32 = jnp.float32
    i32 = jnp.int32
    out_type = [
        jax.ShapeDtypeStruct((NPAD, 32), f32),
        jax.ShapeDtypeStruct((NPAD, 32), f32),
    ]
    scratch = (
        [pltpu.VMEM_SHARED((NPAD, 32), f32)]
        + [pltpu.VMEM((CHE2,), i32)] * (2 * NB2)
        + [pltpu.VMEM((CHE2, 32), f32)] * NB2
        + [pltpu.SemaphoreType.DMA] * NB2
    )
    return pl.kernel(_sc2_body, out_type=out_type, mesh=_mesh(),
                     scratch_types=scratch,
                     compiler_params=pltpu.CompilerParams(
                         use_tc_tiling_on_sc=False))(srcr, dstr, y1a, y1b, z32)


def _dot(a, b):
    return jnp.dot(a, b, preferred_element_type=jnp.float32)


def _sage_dense_body(parts_x, parts_agg, cnt_ref, wl_refs, wr_refs, bl_ref,
                     h_ref, sums_ref):
    i = pl.program_id(0)
    rc = 1.0 / jnp.maximum(cnt_ref[...], 1.0)
    h = bl_ref[...].astype(jnp.float32)
    for a_ref, wl in zip(parts_agg, wl_refs):
        h = h + _dot(a_ref[...] * rc, wl[...])
    for x_ref, wr in zip(parts_x, wr_refs):
        h = h + _dot(x_ref[...], wr[...])
    h_ref[...] = h
    rid = i * RB + lax.broadcasted_iota(jnp.int32, (RB, 1), 0)
    hm = jnp.where(rid < N, h, 0.0)
    ssum = jnp.concatenate(
        [jnp.sum(hm, axis=0, keepdims=True),
         jnp.sum(hm * hm, axis=0, keepdims=True)], axis=0)

    @pl.when(i == 0)
    def _():
        sums_ref[...] = ssum

    @pl.when(i > 0)
    def _():
        sums_ref[...] += ssum


def _sage_dense(xs_parts, agg_parts, cnt2d, wls, wrs, bl):
    nx = len(xs_parts)
    widths = [p.shape[1] for p in xs_parts]

    def body(*refs):
        px = list(refs[0:nx])
        pa = list(refs[nx:2 * nx])
        cnt_ref = refs[2 * nx]
        wl_refs = list(refs[2 * nx + 1:3 * nx + 1])
        wr_refs = list(refs[3 * nx + 1:4 * nx + 1])
        bl_ref = refs[4 * nx + 1]
        h_ref, sums_ref = refs[4 * nx + 2], refs[4 * nx + 3]
        _sage_dense_body(px, pa, cnt_ref, wl_refs, wr_refs, bl_ref,
                         h_ref, sums_ref)

    row_spec = lambda w: pl.BlockSpec((RB, w), lambda i: (i, 0))
    full_spec = lambda a: pl.BlockSpec(a.shape, lambda i: (0, 0))
    in_specs = ([row_spec(w) for w in widths] * 2
                + [pl.BlockSpec((RB, 1), lambda i: (i, 0))]
                + [full_spec(w) for w in wls]
                + [full_spec(w) for w in wrs]
                + [full_spec(bl)])
    return pl.pallas_call(
        body,
        grid=(NBLK,),
        in_specs=in_specs,
        out_specs=[pl.BlockSpec((RB, 64), lambda i: (i, 0)),
                   pl.BlockSpec((2, 64), lambda i: (0, 0))],
        out_shape=[jax.ShapeDtypeStruct((NPAD, 64), jnp.float32),
                   jax.ShapeDtypeStruct((2, 64), jnp.float32)],
    )(*xs_parts, *agg_parts, cnt2d, *wls, *wrs, bl)


def _bn_scale_shift(sums_ref, g_ref, b_ref):
    m = sums_ref[0:1, :] / float(N)
    v = sums_ref[1:2, :] / float(N) - m * m
    sc = g_ref[...] / jnp.sqrt(v + EPS)
    sh = b_ref[...] - m * sc
    return sc, sh


def _bn_relu_split_body(h_ref, sums_ref, g_ref, b_ref, ya_ref, yb_ref):
    sc, sh = _bn_scale_shift(sums_ref, g_ref, b_ref)
    y = jnp.maximum(h_ref[...] * sc + sh, 0.0)
    ya_ref[...] = y[:, :32]
    yb_ref[...] = y[:, 32:]


def _bn_relu_split(h, sums, g, b):
    return pl.pallas_call(
        _bn_relu_split_body,
        grid=(NBLK,),
        in_specs=[pl.BlockSpec((RB, 64), lambda i: (i, 0)),
                  pl.BlockSpec((2, 64), lambda i: (0, 0)),
                  pl.BlockSpec((1, 64), lambda i: (0, 0)),
                  pl.BlockSpec((1, 64), lambda i: (0, 0))],
        out_specs=[pl.BlockSpec((RB, 32), lambda i: (i, 0)),
                   pl.BlockSpec((RB, 32), lambda i: (i, 0))],
        out_shape=[jax.ShapeDtypeStruct((NPAD, 32), jnp.float32),
                   jax.ShapeDtypeStruct((NPAD, 32), jnp.float32)],
    )(h, sums, g, b)


def _bn_relu_pool_head_body(h_ref, sums_ref, g_ref, b_ref, bat_ref, gcnt_ref,
                            wout_ref, bout_ref, out_ref, gsum_ref):
    i = pl.program_id(0)
    sc, sh = _bn_scale_shift(sums_ref, g_ref, b_ref)
    y = jnp.maximum(h_ref[...] * sc + sh, 0.0)
    seg = bat_ref[0]
    gid = lax.broadcasted_iota(jnp.int32, (G, RB), 0)
    oh = jnp.where(gid == seg, 1.0, 0.0)
    p = _dot(oh, y)

    @pl.when(i == 0)
    def _():
        gsum_ref[...] = p

    @pl.when(i > 0)
    def _():
        gsum_ref[...] += p

    @pl.when(i == NBLK - 1)
    def _():
        pooled = gsum_ref[...] / jnp.maximum(gcnt_ref[...], 1.0)
        out_ref[...] = _dot(pooled, wout_ref[...]) + bout_ref[...]


def _bn_relu_pool_head(h, sums, g, b, bat2d, gcnt2d, wout, bout2d):
    return pl.pallas_call(
        _bn_relu_pool_head_body,
        grid=(NBLK,),
        in_specs=[pl.BlockSpec((RB, 64), lambda i: (i, 0)),
                  pl.BlockSpec((2, 64), lambda i: (0, 0)),
                  pl.BlockSpec((1, 64), lambda i: (0, 0)),
                  pl.BlockSpec((1, 64), lambda i: (0, 0)),
                  pl.BlockSpec((1, 1, RB), lambda i: (i, 0, 0)),
                  pl.BlockSpec((G, 1), lambda i: (0, 0)),
                  pl.BlockSpec((64, 2), lambda i: (0, 0)),
                  pl.BlockSpec((1, 2), lambda i: (0, 0))],
        out_specs=pl.BlockSpec((G, 2), lambda i: (0, 0)),
        out_shape=jax.ShapeDtypeStruct((G, 2), jnp.float32),
        scratch_shapes=[pltpu.VMEM((G, 64), jnp.float32)],
    )(h, sums, g, b, bat2d, gcnt2d, wout, bout2d)


def kernel(shape_id, color_id, pos_id, edge_index, batch, shape_emb,
           color_emb, pos_emb, W1l, b1l, W1r, g1, be1, W2l, b2l, W2r, g2,
           be2, Wout, bout):
    i32 = jnp.int32
    f32 = jnp.float32

    src = edge_index[0].astype(i32)
    dst = edge_index[1].astype(i32)
    srcr = jnp.concatenate([src, jnp.zeros((EPAD - E,), i32)])
    dstr = jnp.concatenate([dst, jnp.full((EPAD - E,), N, i32)])
    pad_n = jnp.zeros((NPAD - N,), i32)
    sid = jnp.concatenate([shape_id.astype(i32), pad_n])
    cid = jnp.concatenate([color_id.astype(i32), pad_n])
    pid = jnp.concatenate([pos_id.astype(i32), pad_n])
    batr = jnp.concatenate(
        [batch.astype(i32), jnp.full((NPAD - N,), 520, i32)])

    z16 = jnp.zeros((NT, 16), f32)
    z8 = jnp.zeros((NT, 8), f32)
    z32 = jnp.zeros((NT, 32), f32)
    z1 = jnp.zeros((NT,), f32)
    ones_h = jnp.ones((CHN,), f32)

    xs, xc, xp, aggs, aggc, aggp, cnt, gcnt = _sc1(
        sid, cid, pid, srcr, dstr, batr, shape_emb, color_emb, pos_emb,
        z16, z8, z1, ones_h)

    cnt2d = cnt.reshape(NPAD, 1)
    h1, sums1 = _sage_dense(
        [xs, xc, xp], [aggs, aggc, aggp], cnt2d,
        [W1l[0:16], W1l[16:32], W1l[32:40]],
        [W1r[0:16], W1r[16:32], W1r[32:40]],
        b1l.reshape(1, 64))

    y1a, y1b = _bn_relu_split(h1, sums1, g1.reshape(1, 64), be1.reshape(1, 64))

    agg2a, agg2b = _sc2(srcr, dstr, y1a, y1b, z32)

    h2, sums2 = _sage_dense(
        [y1a, y1b], [agg2a, agg2b], cnt2d,
        [W2l[0:32], W2l[32:64]],
        [W2r[0:32], W2r[32:64]],
        b2l.reshape(1, 64))

    out = _bn_relu_pool_head(
        h2, sums2, g2.reshape(1, 64), be2.reshape(1, 64),
        batr.reshape(NBLK, 1, RB),
        gcnt[:G].reshape(G, 1), Wout, bout.reshape(1, 2))

    return out


# speedup vs baseline: 1.0823x; 1.0823x over previous
import functools

import jax
import jax.numpy as jnp
from jax import lax
from jax.experimental import pallas as pl
from jax.experimental.pallas import tpu as pltpu
import jax.experimental.pallas.tpu_sc as plsc

N = 50000
E = 800000
G = 512
EPS = 1e-5

NPAD = 51200
EPAD = 819200
GP = 640
NC = 2
NS = 16
NT = NPAD // NS
ET = EPAD // NS
CHE1 = 512
NCH1 = ET // CHE1
CHE2 = 256
NB2 = 2
NCH2 = ET // CHE2
CHN = 640
NPCH = NT // CHN
RB = 2048
NBLK = NPAD // RB


def _mesh():
    return plsc.VectorSubcoreMesh(
        core_axis_name="c", subcore_axis_name="s", num_cores=NC, num_subcores=NS)


def _sc1_body(sid, cid, pid, srcr, dstr, batr, semb, cemb, pemb, z16, z8, z1,
              ones_h,
              xs, xc, xp, aggs, aggc, aggp, cnt, gcnt,
              acc16, acc8, cnt_sp, gcnt_sp,
              ei0, ei1, ed0, ed1, er16_0, er16_1, er8_0, er8_1,
              pi_v, pd_v, pr16, pr8, ones_e, ones_p,
              sem0, sem1, semp):
    c = lax.axis_index("c")
    s = lax.axis_index("s")
    nb = s * NT
    eb = s * ET
    ei = (ei0, ei1)
    ed = (ed0, ed1)
    er16 = (er16_0, er16_1)
    er8 = (er8_0, er8_1)
    sems = (sem0, sem1)
    pltpu.sync_copy(ones_h.at[pl.ds(0, CHE1)], ones_e)
    pltpu.sync_copy(ones_h, ones_p)

    pltpu.sync_copy(z16, acc16.at[pl.ds(nb, NT)])

    @pl.when(c == 0)
    def _():
        pltpu.sync_copy(z8, acc8.at[pl.ds(nb, NT)])

    @pl.when(c == 1)
    def _():
        pltpu.sync_copy(z1, cnt_sp.at[pl.ds(nb, NT)])

        @pl.when(s == 0)
        def _():
            pltpu.sync_copy(z1.at[pl.ds(0, GP)], gcnt_sp)

    plsc.subcore_barrier()

    @pl.when(c == 0)
    def _():
        @pl.loop(0, NPCH)
        def _(k):
            o = nb + k * CHN
            pltpu.sync_copy(sid.at[pl.ds(o, CHN)], pi_v)
            pltpu.sync_copy(pid.at[pl.ds(o, CHN)], pd_v)
            pltpu.async_copy(semb.at[pi_v], pr16, semp)
            pltpu.async_copy(pemb.at[pd_v], pr8, semp)
            pltpu.make_async_copy(xs.at[pl.ds(0, CHN)], pr16, semp).wait()
            pltpu.make_async_copy(xp.at[pl.ds(0, CHN)], pr8, semp).wait()
            pltpu.sync_copy(pr16, xs.at[pl.ds(o, CHN)])
            pltpu.sync_copy(pr8, xp.at[pl.ds(o, CHN)])

    @pl.when(c == 1)
    def _():
        @pl.loop(0, NPCH)
        def _(k):
            o = nb + k * CHN
            pltpu.sync_copy(cid.at[pl.ds(o, CHN)], pi_v)
            pltpu.sync_copy(batr.at[pl.ds(o, CHN)], pd_v)
            pltpu.async_copy(cemb.at[pi_v], pr16, semp).wait()
            pltpu.sync_copy(pr16, xc.at[pl.ds(o, CHN)])
            pltpu.sync_copy(ones_p, gcnt_sp.at[pd_v], add=True)

    plsc.subcore_barrier()

    @pl.when(c == 0)
    def _():
        for b in range(2):
            pltpu.sync_copy(srcr.at[pl.ds(eb + b * CHE1, CHE1)], ei[b])
            pltpu.sync_copy(dstr.at[pl.ds(eb + b * CHE1, CHE1)], ed[b])
            pltpu.async_copy(xs.at[ei[b]], er16[b], sems[b])
            pltpu.async_copy(xp.at[ei[b]], er8[b], sems[b])

        @pl.loop(0, NCH1, step=2)
        def _(k):
            for b in range(2):
                pltpu.make_async_copy(
                    xs.at[pl.ds(0, CHE1)], er16[b], sems[b]).wait()
                pltpu.make_async_copy(
                    xp.at[pl.ds(0, CHE1)], er8[b], sems[b]).wait()
                pltpu.sync_copy(er16[b], acc16.at[ed[b]], add=True)
                pltpu.sync_copy(er8[b], acc8.at[ed[b]], add=True)

                @pl.when(k + (b + 2) < NCH1)
                def _():
                    o = eb + (k + (b + 2)) * CHE1
                    pltpu.sync_copy(srcr.at[pl.ds(o, CHE1)], ei[b])
                    pltpu.sync_copy(dstr.at[pl.ds(o, CHE1)], ed[b])
                    pltpu.async_copy(xs.at[ei[b]], er16[b], sems[b])
                    pltpu.async_copy(xp.at[ei[b]], er8[b], sems[b])

    @pl.when(c == 1)
    def _():
        for b in range(2):
            pltpu.sync_copy(srcr.at[pl.ds(eb + b * CHE1, CHE1)], ei[b])
            pltpu.sync_copy(dstr.at[pl.ds(eb + b * CHE1, CHE1)], ed[b])
            pltpu.async_copy(xc.at[ei[b]], er16[b], sems[b])

        @pl.loop(0, NCH1, step=2)
        def _(k):
            for b in range(2):
                pltpu.make_async_copy(
                    xc.at[pl.ds(0, CHE1)], er16[b], sems[b]).wait()
                pltpu.sync_copy(er16[b], acc16.at[ed[b]], add=True)
                pltpu.sync_copy(ones_e, cnt_sp.at[ed[b]], add=True)

                @pl.when(k + (b + 2) < NCH1)
                def _():
                    o = eb + (k + (b + 2)) * CHE1
                    pltpu.sync_copy(srcr.at[pl.ds(o, CHE1)], ei[b])
                    pltpu.sync_copy(dstr.at[pl.ds(o, CHE1)], ed[b])
                    pltpu.async_copy(xc.at[ei[b]], er16[b], sems[b])

    plsc.subcore_barrier()

    @pl.when(c == 0)
    def _():
        pltpu.sync_copy(acc16.at[pl.ds(nb, NT)], aggs.at[pl.ds(nb, NT)])
        pltpu.sync_copy(acc8.at[pl.ds(nb, NT)], aggp.at[pl.ds(nb, NT)])

    @pl.when(c == 1)
    def _():
        pltpu.sync_copy(acc16.at[pl.ds(nb, NT)], aggc.at[pl.ds(nb, NT)])
        pltpu.sync_copy(cnt_sp.at[pl.ds(nb, NT)], cnt.at[pl.ds(nb, NT)])

        @pl.when(s == 0)
        def _():
            pltpu.sync_copy(gcnt_sp, gcnt)


def _sc1(sid, cid, pid, srcr, dstr, batr, semb, cemb, pemb, z16, z8, z1,
         ones_h):
    f32 = jnp.float32
    i32 = jnp.int32
    out_type = [
        jax.ShapeDtypeStruct((NPAD, 16), f32),
        jax.ShapeDtypeStruct((NPAD, 16), f32),
        jax.ShapeDtypeStruct((NPAD, 8), f32),
        jax.ShapeDtypeStruct((NPAD, 16), f32),
        jax.ShapeDtypeStruct((NPAD, 16), f32),
        jax.ShapeDtypeStruct((NPAD, 8), f32),
        jax.ShapeDtypeStruct((NPAD,), f32),
        jax.ShapeDtypeStruct((GP,), f32),
    ]
    scratch = [
        pltpu.VMEM_SHARED((NPAD, 16), f32),
        pltpu.VMEM_SHARED((NPAD, 8), f32),
        pltpu.VMEM_SHARED((NPAD,), f32),
        pltpu.VMEM_SHARED((GP,), f32),
        pltpu.VMEM((CHE1,), i32),
        pltpu.VMEM((CHE1,), i32),
        pltpu.VMEM((CHE1,), i32),
        pltpu.VMEM((CHE1,), i32),
        pltpu.VMEM((CHE1, 16), f32),
        pltpu.VMEM((CHE1, 16), f32),
        pltpu.VMEM((CHE1, 8), f32),
        pltpu.VMEM((CHE1, 8), f32),
        pltpu.VMEM((CHN,), i32),
        pltpu.VMEM((CHN,), i32),
        pltpu.VMEM((CHN, 16), f32),
        pltpu.VMEM((CHN, 8), f32),
        pltpu.VMEM((CHE1,), f32),
        pltpu.VMEM((CHN,), f32),
        pltpu.SemaphoreType.DMA,
        pltpu.SemaphoreType.DMA,
        pltpu.SemaphoreType.DMA,
    ]
    return pl.kernel(_sc1_body, out_type=out_type, mesh=_mesh(),
                     scratch_types=scratch,
                     compiler_params=pltpu.CompilerParams(
                         use_tc_tiling_on_sc=False))(
        sid, cid, pid, srcr, dstr, batr, semb, cemb, pemb, z16, z8, z1,
        ones_h)


def _sc2_body(srcr, dstr, y1a, y1b, z32, agg2a, agg2b, acc32, *rest):
    c = lax.axis_index("c")
    s = lax.axis_index("s")
    nb = s * NT
    eb = s * ET
    ei = rest[0:NB2]
    ed = rest[NB2:2 * NB2]
    er32 = rest[2 * NB2:3 * NB2]
    sems = rest[3 * NB2:4 * NB2]

    pltpu.sync_copy(z32, acc32.at[pl.ds(nb, NT)])
    plsc.subcore_barrier()

    def edge_loop(table):
        for b in range(NB2):
            pltpu.sync_copy(srcr.at[pl.ds(eb + b * CHE2, CHE2)], ei[b])
            pltpu.sync_copy(dstr.at[pl.ds(eb + b * CHE2, CHE2)], ed[b])
            pltpu.async_copy(table.at[ei[b]], er32[b], sems[b])

        @pl.loop(0, NCH2, step=NB2)
        def _(k):
            for b in range(NB2):
                pltpu.make_async_copy(
                    table.at[pl.ds(0, CHE2)], er32[b], sems[b]).wait()
                pltpu.sync_copy(er32[b], acc32.at[ed[b]], add=True)

                @pl.when(k + (b + NB2) < NCH2)
                def _():
                    o = eb + (k + (b + NB2)) * CHE2
                    pltpu.sync_copy(srcr.at[pl.ds(o, CHE2)], ei[b])
                    pltpu.sync_copy(dstr.at[pl.ds(o, CHE2)], ed[b])
                    pltpu.async_copy(table.at[ei[b]], er32[b], sems[b])

    @pl.when(c == 0)
    def _():
        edge_loop(y1a)

    @pl.when(c == 1)
    def _():
        edge_loop(y1b)

    plsc.subcore_barrier()

    @pl.when(c == 0)
    def _():
        pltpu.sync_copy(acc32.at[pl.ds(nb, NT)], agg2a.at[pl.ds(nb, NT)])

    @pl.when(c == 1)
    def _():
        pltpu.sync_copy(acc32.at[pl.ds(nb, NT)], agg2b.at[pl.ds(nb, NT)])


def _sc2(srcr, dstr, y1a, y1b, z32):
    f32 = jnp.float32
    i32 = jnp.int32
    out_type = [
        jax.ShapeDtypeStruct((NPAD, 32), f32),
        jax.ShapeDtypeStruct((NPAD, 32), f32),
    ]
    scratch = (
        [pltpu.VMEM_SHARED((NPAD, 32), f32)]
        + [pltpu.VMEM((CHE2,), i32)] * (2 * NB2)
        + [pltpu.VMEM((CHE2, 32), f32)] * NB2
        + [pltpu.SemaphoreType.DMA] * NB2
    )
    return pl.kernel(_sc2_body, out_type=out_type, mesh=_mesh(),
                     scratch_types=scratch,
                     compiler_params=pltpu.CompilerParams(
                         use_tc_tiling_on_sc=False))(srcr, dstr, y1a, y1b, z32)


def _dot(a, b):
    return jnp.dot(a, b, preferred_element_type=jnp.float32)


NF = NPAD // 8
RBF = NF // NBLK
NRF = N // 8


def _sage_dense_folded_body(x16w, xw, a16w, aw, rc_ref, w_ref, bl_ref,
                            h_ref, sums_ref):
    i = pl.program_id(0)
    hs = []
    for k in range(8):
        rc = rc_ref[:, 16 * k:16 * k + 1]
        xp = [r[:, (r.shape[1] // 8) * k:(r.shape[1] // 8) * (k + 1)]
              for r in x16w + xw]
        ap = [r[:, (r.shape[1] // 8) * k:(r.shape[1] // 8) * (k + 1)]
              for r in a16w + aw]
        cat = jnp.concatenate(xp + [a * rc for a in ap], axis=1)
        hs.append(_dot(cat, w_ref[...]) + bl_ref[...])
    h = jnp.concatenate(hs, axis=1)
    h_ref[...] = h
    rid = i * RBF + lax.broadcasted_iota(jnp.int32, (RBF, 1), 0)
    hm = jnp.where(rid < NRF, h, 0.0)
    ssum = jnp.concatenate(
        [jnp.sum(hm, axis=0, keepdims=True),
         jnp.sum(hm * hm, axis=0, keepdims=True)], axis=0)

    @pl.when(i == 0)
    def _():
        sums_ref[...] = ssum

    @pl.when(i > 0)
    def _():
        sums_ref[...] += ssum


def _sage_dense_folded(x16, xodd, a16, aodd, rcf, wcat, blf):
    n16 = len(x16)
    nod = len(xodd)

    def body(*refs):
        p = 0
        x16r = list(refs[p:p + n16]); p += n16
        xor_ = list(refs[p:p + nod]); p += nod
        a16r = list(refs[p:p + n16]); p += n16
        aor = list(refs[p:p + nod]); p += nod
        rc_ref, w_ref, bl_ref = refs[p], refs[p + 1], refs[p + 2]
        h_ref, sums_ref = refs[p + 3], refs[p + 4]
        _sage_dense_folded_body(x16r, xor_, a16r, aor, rc_ref, w_ref, bl_ref,
                                h_ref, sums_ref)

    row_spec = lambda a: pl.BlockSpec((RBF, a.shape[1]), lambda i: (i, 0))
    full_spec = lambda a: pl.BlockSpec(a.shape, lambda i: (0, 0))
    arrs = x16 + xodd + a16 + aodd
    in_specs = ([row_spec(a) for a in arrs]
                + [row_spec(rcf), full_spec(wcat), full_spec(blf)])
    return pl.pallas_call(
        body,
        grid=(NBLK,),
        in_specs=in_specs,
        out_specs=[pl.BlockSpec((RBF, 512), lambda i: (i, 0)),
                   pl.BlockSpec((2, 512), lambda i: (0, 0))],
        out_shape=[jax.ShapeDtypeStruct((NF, 512), jnp.float32),
                   jax.ShapeDtypeStruct((2, 512), jnp.float32)],
    )(*arrs, rcf, wcat, blf)


def _bn_relu_folded_body(h_ref, sc_ref, sh_ref, y_ref):
    y_ref[...] = jnp.maximum(h_ref[...] * sc_ref[...] + sh_ref[...], 0.0)


def _bn_relu_folded(hf, scf, shf):
    return pl.pallas_call(
        _bn_relu_folded_body,
        grid=(NBLK,),
        in_specs=[pl.BlockSpec((RBF, 512), lambda i: (i, 0)),
                  pl.BlockSpec((1, 512), lambda i: (0, 0)),
                  pl.BlockSpec((1, 512), lambda i: (0, 0))],
        out_specs=pl.BlockSpec((RBF, 512), lambda i: (i, 0)),
        out_shape=jax.ShapeDtypeStruct((NF, 512), jnp.float32),
    )(hf, scf, shf)


def _pool_head_folded_body(h_ref, sc_ref, sh_ref, bat_ref, gcnt_ref,
                           wout_ref, bout_ref, out_ref, gsum_ref):
    i = pl.program_id(0)
    y = jnp.maximum(h_ref[...] * sc_ref[...] + sh_ref[...], 0.0)
    p = jnp.zeros((G, 64), jnp.float32)
    gid = lax.broadcasted_iota(jnp.int32, (G, RBF), 0)
    for k in range(8):
        seg = bat_ref[0, k:k + 1, :]
        oh = jnp.where(gid == seg, 1.0, 0.0)
        p = p + _dot(oh, y[:, 64 * k:64 * (k + 1)])

    @pl.when(i == 0)
    def _():
        gsum_ref[...] = p

    @pl.when(i > 0)
    def _():
        gsum_ref[...] += p

    @pl.when(i == NBLK - 1)
    def _():
        pooled = gsum_ref[...] / jnp.maximum(gcnt_ref[...], 1.0)
        out_ref[...] = _dot(pooled, wout_ref[...]) + bout_ref[...]


def _pool_head_folded(hf, scf, shf, batf, gcnt2d, wout, bout2d):
    return pl.pallas_call(
        _pool_head_folded_body,
        grid=(NBLK,),
        in_specs=[pl.BlockSpec((RBF, 512), lambda i: (i, 0)),
                  pl.BlockSpec((1, 512), lambda i: (0, 0)),
                  pl.BlockSpec((1, 512), lambda i: (0, 0)),
                  pl.BlockSpec((1, 8, RBF), lambda i: (i, 0, 0)),
                  pl.BlockSpec((G, 1), lambda i: (0, 0)),
                  pl.BlockSpec((64, 2), lambda i: (0, 0)),
                  pl.BlockSpec((1, 2), lambda i: (0, 0))],
        out_specs=pl.BlockSpec((G, 2), lambda i: (0, 0)),
        out_shape=jax.ShapeDtypeStruct((G, 2), jnp.float32),
        scratch_shapes=[pltpu.VMEM((G, 64), jnp.float32)],
    )(hf, scf, shf, batf, gcnt2d, wout, bout2d)


def _bn_scale_shift_host(sums512, g, b):
    s = sums512.reshape(2, 8, 64).sum(axis=1)
    m = s[0] / float(N)
    v = s[1] / float(N) - m * m
    sc = g / jnp.sqrt(v + EPS)
    sh = b - m * sc
    return jnp.tile(sc, 8).reshape(1, 512), jnp.tile(sh, 8).reshape(1, 512)


def kernel(shape_id, color_id, pos_id, edge_index, batch, shape_emb,
           color_emb, pos_emb, W1l, b1l, W1r, g1, be1, W2l, b2l, W2r, g2,
           be2, Wout, bout):
    i32 = jnp.int32
    f32 = jnp.float32

    src = edge_index[0].astype(i32)
    dst = edge_index[1].astype(i32)
    srcr = jnp.concatenate([src, jnp.zeros((EPAD - E,), i32)])
    dstr = jnp.concatenate([dst, jnp.full((EPAD - E,), N, i32)])
    pad_n = jnp.zeros((NPAD - N,), i32)
    sid = jnp.concatenate([shape_id.astype(i32), pad_n])
    cid = jnp.concatenate([color_id.astype(i32), pad_n])
    pid = jnp.concatenate([pos_id.astype(i32), pad_n])
    batr = jnp.concatenate(
        [batch.astype(i32), jnp.full((NPAD - N,), 520, i32)])

    z16 = jnp.zeros((NT, 16), f32)
    z8 = jnp.zeros((NT, 8), f32)
    z32 = jnp.zeros((NT, 32), f32)
    z1 = jnp.zeros((NT,), f32)
    ones_h = jnp.ones((CHN,), f32)

    xs, xc, xp, aggs, aggc, aggp, cnt, gcnt = _sc1(
        sid, cid, pid, srcr, dstr, batr, shape_emb, color_emb, pos_emb,
        z16, z8, z1, ones_h)

    rcf = jnp.broadcast_to(
        (1.0 / jnp.maximum(cnt, 1.0)).reshape(NPAD, 1), (NPAD, 16)
    ).reshape(NF, 128)
    h1f, sums1 = _sage_dense_folded(
        [xs.reshape(NF, 128), xc.reshape(NF, 128)], [xp.reshape(NF, 64)],
        [aggs.reshape(NF, 128), aggc.reshape(NF, 128)], [aggp.reshape(NF, 64)],
        rcf, jnp.concatenate([W1r, W1l], axis=0), b1l.reshape(1, 64))

    sc1v, sh1v = _bn_scale_shift_host(sums1, g1, be1)
    y1f = _bn_relu_folded(h1f, sc1v, sh1v)
    y1 = y1f.reshape(NPAD, 64)
    y1a = y1[:, :32]
    y1b = y1[:, 32:]

    agg2a, agg2b = _sc2(srcr, dstr, y1a, y1b, z32)

    h2f, sums2 = _sage_dense_folded(
        [], [y1a.reshape(NF, 256), y1b.reshape(NF, 256)],
        [], [agg2a.reshape(NF, 256), agg2b.reshape(NF, 256)],
        rcf, jnp.concatenate([W2r, W2l], axis=0), b2l.reshape(1, 64))

    sc2v, sh2v = _bn_scale_shift_host(sums2, g2, be2)
    batf = batr.reshape(NBLK, RBF, 8).transpose(0, 2, 1)
    out = _pool_head_folded(
        h2f, sc2v, sh2v, batf,
        gcnt[:G].reshape(G, 1), Wout, bout.reshape(1, 2))

    return out


# speedup vs baseline: 1.1705x; 1.0816x over previous
import functools

import jax
import jax.numpy as jnp
from jax import lax
from jax.experimental import pallas as pl
from jax.experimental.pallas import tpu as pltpu
import jax.experimental.pallas.tpu_sc as plsc

N = 50000
E = 800000
G = 512
EPS = 1e-5

NPAD = 51200
EPAD = 819200
GP = 640
NC = 2
NS = 16
NT = NPAD // NS
ET = EPAD // NS
CHE1 = 512
NCH1 = ET // CHE1
CHE2 = 256
NB2 = 2
NCH2 = ET // CHE2
CHN = 640
NPCH = NT // CHN
RB = 2048
NBLK = NPAD // RB


def _mesh():
    return plsc.VectorSubcoreMesh(
        core_axis_name="c", subcore_axis_name="s", num_cores=NC, num_subcores=NS)


def _sc1_body(sid, cid, pid, srcr, dstr, batr, semb, cemb, pemb, z16, z8, z1,
              ones_h,
              xs, xc, xp, aggs, aggc, aggp, cnt, gcnt,
              acc16, acc8, cnt_sp, gcnt_sp,
              ei0, ei1, ed0, ed1, er16_0, er16_1, er8_0, er8_1,
              pi_v, pd_v, pr16, pr8, ones_e, ones_p,
              sem0, sem1, semp):
    c = lax.axis_index("c")
    s = lax.axis_index("s")
    nb = s * NT
    eb = s * ET
    ei = (ei0, ei1)
    ed = (ed0, ed1)
    er16 = (er16_0, er16_1)
    er8 = (er8_0, er8_1)
    sems = (sem0, sem1)
    pltpu.sync_copy(ones_h.at[pl.ds(0, CHE1)], ones_e)
    pltpu.sync_copy(ones_h, ones_p)

    pltpu.sync_copy(z16, acc16.at[pl.ds(nb, NT)])

    @pl.when(c == 0)
    def _():
        pltpu.sync_copy(z8, acc8.at[pl.ds(nb, NT)])

    @pl.when(c == 1)
    def _():
        pltpu.sync_copy(z1, cnt_sp.at[pl.ds(nb, NT)])

        @pl.when(s == 0)
        def _():
            pltpu.sync_copy(z1.at[pl.ds(0, GP)], gcnt_sp)

    plsc.subcore_barrier()

    @pl.when(c == 0)
    def _():
        @pl.loop(0, NPCH)
        def _(k):
            o = nb + k * CHN
            pltpu.sync_copy(sid.at[pl.ds(o, CHN)], pi_v)
            pltpu.sync_copy(pid.at[pl.ds(o, CHN)], pd_v)
            pltpu.async_copy(semb.at[pi_v], pr16, semp)
            pltpu.async_copy(pemb.at[pd_v], pr8, semp)
            pltpu.make_async_copy(xs.at[pl.ds(0, CHN)], pr16, semp).wait()
            pltpu.make_async_copy(xp.at[pl.ds(0, CHN)], pr8, semp).wait()
            pltpu.sync_copy(pr16, xs.at[pl.ds(o, CHN)])
            pltpu.sync_copy(pr8, xp.at[pl.ds(o, CHN)])

    @pl.when(c == 1)
    def _():
        @pl.loop(0, NPCH)
        def _(k):
            o = nb + k * CHN
            pltpu.sync_copy(cid.at[pl.ds(o, CHN)], pi_v)
            pltpu.sync_copy(batr.at[pl.ds(o, CHN)], pd_v)
            pltpu.async_copy(cemb.at[pi_v], pr16, semp).wait()
            pltpu.sync_copy(pr16, xc.at[pl.ds(o, CHN)])
            pltpu.sync_copy(ones_p, gcnt_sp.at[pd_v], add=True)

    plsc.subcore_barrier()

    @pl.when(c == 0)
    def _():
        for b in range(2):
            pltpu.sync_copy(srcr.at[pl.ds(eb + b * CHE1, CHE1)], ei[b])
            pltpu.sync_copy(dstr.at[pl.ds(eb + b * CHE1, CHE1)], ed[b])
            pltpu.async_copy(xs.at[ei[b]], er16[b], sems[b])
            pltpu.async_copy(xp.at[ei[b]], er8[b], sems[b])

        @pl.loop(0, NCH1, step=2)
        def _(k):
            for b in range(2):
                pltpu.make_async_copy(
                    xs.at[pl.ds(0, CHE1)], er16[b], sems[b]).wait()
                pltpu.make_async_copy(
                    xp.at[pl.ds(0, CHE1)], er8[b], sems[b]).wait()
                pltpu.sync_copy(er16[b], acc16.at[ed[b]], add=True)
                pltpu.sync_copy(er8[b], acc8.at[ed[b]], add=True)

                @pl.when(k + (b + 2) < NCH1)
                def _():
                    o = eb + (k + (b + 2)) * CHE1
                    pltpu.sync_copy(srcr.at[pl.ds(o, CHE1)], ei[b])
                    pltpu.sync_copy(dstr.at[pl.ds(o, CHE1)], ed[b])
                    pltpu.async_copy(xs.at[ei[b]], er16[b], sems[b])
                    pltpu.async_copy(xp.at[ei[b]], er8[b], sems[b])

    @pl.when(c == 1)
    def _():
        for b in range(2):
            pltpu.sync_copy(srcr.at[pl.ds(eb + b * CHE1, CHE1)], ei[b])
            pltpu.sync_copy(dstr.at[pl.ds(eb + b * CHE1, CHE1)], ed[b])
            pltpu.async_copy(xc.at[ei[b]], er16[b], sems[b])

        @pl.loop(0, NCH1, step=2)
        def _(k):
            for b in range(2):
                pltpu.make_async_copy(
                    xc.at[pl.ds(0, CHE1)], er16[b], sems[b]).wait()
                pltpu.sync_copy(er16[b], acc16.at[ed[b]], add=True)
                pltpu.sync_copy(ones_e, cnt_sp.at[ed[b]], add=True)

                @pl.when(k + (b + 2) < NCH1)
                def _():
                    o = eb + (k + (b + 2)) * CHE1
                    pltpu.sync_copy(srcr.at[pl.ds(o, CHE1)], ei[b])
                    pltpu.sync_copy(dstr.at[pl.ds(o, CHE1)], ed[b])
                    pltpu.async_copy(xc.at[ei[b]], er16[b], sems[b])

    plsc.subcore_barrier()

    @pl.when(c == 0)
    def _():
        pltpu.sync_copy(acc16.at[pl.ds(nb, NT)], aggs.at[pl.ds(nb, NT)])
        pltpu.sync_copy(acc8.at[pl.ds(nb, NT)], aggp.at[pl.ds(nb, NT)])

    @pl.when(c == 1)
    def _():
        pltpu.sync_copy(acc16.at[pl.ds(nb, NT)], aggc.at[pl.ds(nb, NT)])
        pltpu.sync_copy(cnt_sp.at[pl.ds(nb, NT)], cnt.at[pl.ds(nb, NT)])

        @pl.when(s == 0)
        def _():
            pltpu.sync_copy(gcnt_sp, gcnt)


def _sc1(sid, cid, pid, srcr, dstr, batr, semb, cemb, pemb, z16, z8, z1,
         ones_h):
    f32 = jnp.float32
    i32 = jnp.int32
    out_type = [
        jax.ShapeDtypeStruct((NPAD, 16), f32),
        jax.ShapeDtypeStruct((NPAD, 16), f32),
        jax.ShapeDtypeStruct((NPAD, 8), f32),
        jax.ShapeDtypeStruct((NPAD, 16), f32),
        jax.ShapeDtypeStruct((NPAD, 16), f32),
        jax.ShapeDtypeStruct((NPAD, 8), f32),
        jax.ShapeDtypeStruct((NPAD,), f32),
        jax.ShapeDtypeStruct((GP,), f32),
    ]
    scratch = [
        pltpu.VMEM_SHARED((NPAD, 16), f32),
        pltpu.VMEM_SHARED((NPAD, 8), f32),
        pltpu.VMEM_SHARED((NPAD,), f32),
        pltpu.VMEM_SHARED((GP,), f32),
        pltpu.VMEM((CHE1,), i32),
        pltpu.VMEM((CHE1,), i32),
        pltpu.VMEM((CHE1,), i32),
        pltpu.VMEM((CHE1,), i32),
        pltpu.VMEM((CHE1, 16), f32),
        pltpu.VMEM((CHE1, 16), f32),
        pltpu.VMEM((CHE1, 8), f32),
        pltpu.VMEM((CHE1, 8), f32),
        pltpu.VMEM((CHN,), i32),
        pltpu.VMEM((CHN,), i32),
        pltpu.VMEM((CHN, 16), f32),
        pltpu.VMEM((CHN, 8), f32),
        pltpu.VMEM((CHE1,), f32),
        pltpu.VMEM((CHN,), f32),
        pltpu.SemaphoreType.DMA,
        pltpu.SemaphoreType.DMA,
        pltpu.SemaphoreType.DMA,
    ]
    return pl.kernel(_sc1_body, out_type=out_type, mesh=_mesh(),
                     scratch_types=scratch,
                     compiler_params=pltpu.CompilerParams(
                         use_tc_tiling_on_sc=False))(
        sid, cid, pid, srcr, dstr, batr, semb, cemb, pemb, z16, z8, z1,
        ones_h)


def _sc2_body(srcr, dstr, y1a, y1b, z32, agg2a, agg2b, acc32, *rest):
    c = lax.axis_index("c")
    s = lax.axis_index("s")
    nb = s * NT
    eb = s * ET
    ei = rest[0:NB2]
    ed = rest[NB2:2 * NB2]
    er32 = rest[2 * NB2:3 * NB2]
    sems = rest[3 * NB2:4 * NB2]

    pltpu.sync_copy(z32, acc32.at[pl.ds(nb, NT)])
    plsc.subcore_barrier()

    def edge_loop(table):
        for b in range(NB2):
            pltpu.sync_copy(srcr.at[pl.ds(eb + b * CHE2, CHE2)], ei[b])
            pltpu.sync_copy(dstr.at[pl.ds(eb + b * CHE2, CHE2)], ed[b])
            pltpu.async_copy(table.at[ei[b]], er32[b], sems[b])

        @pl.loop(0, NCH2, step=NB2)
        def _(k):
            for b in range(NB2):
                pltpu.make_async_copy(
                    table.at[pl.ds(0, CHE2)], er32[b], sems[b]).wait()
                pltpu.sync_copy(er32[b], acc32.at[ed[b]], add=True)

                @pl.when(k + (b + NB2) < NCH2)
                def _():
                    o = eb + (k + (b + NB2)) * CHE2
                    pltpu.sync_copy(srcr.at[pl.ds(o, CHE2)], ei[b])
                    pltpu.sync_copy(dstr.at[pl.ds(o, CHE2)], ed[b])
                    pltpu.async_copy(table.at[ei[b]], er32[b], sems[b])

    @pl.when(c == 0)
    def _():
        edge_loop(y1a)

    @pl.when(c == 1)
    def _():
        edge_loop(y1b)

    plsc.subcore_barrier()

    @pl.when(c == 0)
    def _():
        pltpu.sync_copy(acc32.at[pl.ds(nb, NT)], agg2a.at[pl.ds(nb, NT)])

    @pl.when(c == 1)
    def _():
        pltpu.sync_copy(acc32.at[pl.ds(nb, NT)], agg2b.at[pl.ds(nb, NT)])


def _sc2(srcr, dstr, y1a, y1b, z32):
    f32 = jnp.float32
    i32 = jnp.int32
    out_type = [
        jax.ShapeDtypeStruct((NPAD, 32), f32),
        jax.ShapeDtypeStruct((NPAD, 32), f32),
    ]
    scratch = (
        [pltpu.VMEM_SHARED((NPAD, 32), f32)]
        + [pltpu.VMEM((CHE2,), i32)] * (2 * NB2)
        + [pltpu.VMEM((CHE2, 32), f32)] * NB2
        + [pltpu.SemaphoreType.DMA] * NB2
    )
    return pl.kernel(_sc2_body, out_type=out_type, mesh=_mesh(),
                     scratch_types=scratch,
                     compiler_params=pltpu.CompilerParams(
                         use_tc_tiling_on_sc=False))(srcr, dstr, y1a, y1b, z32)


def _dot(a, b):
    return jnp.dot(a, b, preferred_element_type=jnp.float32)


NF = NPAD // 8
RBF = NF // NBLK
NRF = N // 8


def _sage_dense_folded_body(x16w, xw, a16w, aw, rc_ref, w_ref, bl_ref,
                            h_ref, sums_ref):
    i = pl.program_id(0)
    hs = []
    for k in range(8):
        rc = rc_ref[:, 16 * k:16 * k + 1]
        xp = [r[:, (r.shape[1] // 8) * k:(r.shape[1] // 8) * (k + 1)]
              for r in x16w + xw]
        ap = [r[:, (r.shape[1] // 8) * k:(r.shape[1] // 8) * (k + 1)]
              for r in a16w + aw]
        cat = jnp.concatenate(xp + [a * rc for a in ap], axis=1)
        hs.append(_dot(cat, w_ref[...]) + bl_ref[...])
    h = jnp.concatenate(hs, axis=1)
    h_ref[...] = h
    rid = i * RBF + lax.broadcasted_iota(jnp.int32, (RBF, 1), 0)
    hm = jnp.where(rid < NRF, h, 0.0)
    ssum = jnp.concatenate(
        [jnp.sum(hm, axis=0, keepdims=True),
         jnp.sum(hm * hm, axis=0, keepdims=True)], axis=0)

    @pl.when(i == 0)
    def _():
        sums_ref[...] = ssum

    @pl.when(i > 0)
    def _():
        sums_ref[...] += ssum


def _sage_dense_folded(x16, xodd, a16, aodd, rcf, wcat, blf):
    n16 = len(x16)
    nod = len(xodd)

    def body(*refs):
        p = 0
        x16r = list(refs[p:p + n16]); p += n16
        xor_ = list(refs[p:p + nod]); p += nod
        a16r = list(refs[p:p + n16]); p += n16
        aor = list(refs[p:p + nod]); p += nod
        rc_ref, w_ref, bl_ref = refs[p], refs[p + 1], refs[p + 2]
        h_ref, sums_ref = refs[p + 3], refs[p + 4]
        _sage_dense_folded_body(x16r, xor_, a16r, aor, rc_ref, w_ref, bl_ref,
                                h_ref, sums_ref)

    row_spec = lambda a: pl.BlockSpec((RBF, a.shape[1]), lambda i: (i, 0))
    full_spec = lambda a: pl.BlockSpec(a.shape, lambda i: (0, 0))
    arrs = x16 + xodd + a16 + aodd
    in_specs = ([row_spec(a) for a in arrs]
                + [row_spec(rcf), full_spec(wcat), full_spec(blf)])
    return pl.pallas_call(
        body,
        grid=(NBLK,),
        in_specs=in_specs,
        out_specs=[pl.BlockSpec((RBF, 512), lambda i: (i, 0)),
                   pl.BlockSpec((2, 512), lambda i: (0, 0))],
        out_shape=[jax.ShapeDtypeStruct((NF, 512), jnp.float32),
                   jax.ShapeDtypeStruct((2, 512), jnp.float32)],
    )(*arrs, rcf, wcat, blf)


def _bn_relu_split_folded_body(h_ref, sc_ref, sh_ref, ya_ref, yb_ref):
    y = jnp.maximum(h_ref[...] * sc_ref[...] + sh_ref[...], 0.0)
    for j in range(2):
        ya_ref[j] = jnp.concatenate(
            [y[:, (4 * j + m) * 64:(4 * j + m) * 64 + 32] for m in range(4)],
            axis=1)
        yb_ref[j] = jnp.concatenate(
            [y[:, (4 * j + m) * 64 + 32:(4 * j + m + 1) * 64]
             for m in range(4)], axis=1)


def _bn_relu_split_folded(hf, scf, shf):
    return pl.pallas_call(
        _bn_relu_split_folded_body,
        grid=(NBLK,),
        in_specs=[pl.BlockSpec((RBF, 512), lambda i: (i, 0)),
                  pl.BlockSpec((1, 512), lambda i: (0, 0)),
                  pl.BlockSpec((1, 512), lambda i: (0, 0))],
        out_specs=[pl.BlockSpec((2, RBF, 128), lambda i: (0, i, 0)),
                   pl.BlockSpec((2, RBF, 128), lambda i: (0, i, 0))],
        out_shape=[jax.ShapeDtypeStruct((2, NF, 128), jnp.float32),
                   jax.ShapeDtypeStruct((2, NF, 128), jnp.float32)],
    )(hf, scf, shf)


def _sage2_folded_body(ya_ref, yb_ref, a2a_ref, a2b_ref, rc_ref, w_ref,
                       bl_ref, h_ref, sums_ref):
    i = pl.program_id(0)
    hs = []
    for k in range(8):
        j, m = divmod(k, 4)
        rc = rc_ref[:, 16 * k:16 * k + 1]
        y_k = jnp.concatenate(
            [ya_ref[j][:, 32 * m:32 * m + 32],
             yb_ref[j][:, 32 * m:32 * m + 32]], axis=1)
        a_k = jnp.concatenate(
            [a2a_ref[:, 32 * k:32 * k + 32],
             a2b_ref[:, 32 * k:32 * k + 32]], axis=1) * rc
        cat = jnp.concatenate([y_k, a_k], axis=1)
        hs.append(_dot(cat, w_ref[...]) + bl_ref[...])
    h = jnp.concatenate(hs, axis=1)
    h_ref[...] = h
    rid = i * RBF + lax.broadcasted_iota(jnp.int32, (RBF, 1), 0)
    hm = jnp.where(rid < NRF, h, 0.0)
    ssum = jnp.concatenate(
        [jnp.sum(hm, axis=0, keepdims=True),
         jnp.sum(hm * hm, axis=0, keepdims=True)], axis=0)

    @pl.when(i == 0)
    def _():
        sums_ref[...] = ssum

    @pl.when(i > 0)
    def _():
        sums_ref[...] += ssum


def _sage2_folded(ya, yb, a2af, a2bf, rcf, wcat, blf):
    full_spec = lambda a: pl.BlockSpec(a.shape, lambda i: (0, 0))
    return pl.pallas_call(
        _sage2_folded_body,
        grid=(NBLK,),
        in_specs=[pl.BlockSpec((2, RBF, 128), lambda i: (0, i, 0)),
                  pl.BlockSpec((2, RBF, 128), lambda i: (0, i, 0)),
                  pl.BlockSpec((RBF, 256), lambda i: (i, 0)),
                  pl.BlockSpec((RBF, 256), lambda i: (i, 0)),
                  pl.BlockSpec((RBF, 128), lambda i: (i, 0)),
                  full_spec(wcat), full_spec(blf)],
        out_specs=[pl.BlockSpec((RBF, 512), lambda i: (i, 0)),
                   pl.BlockSpec((2, 512), lambda i: (0, 0))],
        out_shape=[jax.ShapeDtypeStruct((NF, 512), jnp.float32),
                   jax.ShapeDtypeStruct((2, 512), jnp.float32)],
    )(ya, yb, a2af, a2bf, rcf, wcat, blf)


def _pool_head_folded_body(h_ref, sc_ref, sh_ref, bat_ref, gcnt_ref,
                           wout_ref, bout_ref, out_ref, gsum_ref):
    i = pl.program_id(0)
    y = jnp.maximum(h_ref[...] * sc_ref[...] + sh_ref[...], 0.0)
    p = jnp.zeros((G, 64), jnp.float32)
    gid = lax.broadcasted_iota(jnp.int32, (G, RBF), 0)
    for k in range(8):
        seg = bat_ref[0, k:k + 1, :]
        oh = jnp.where(gid == seg, 1.0, 0.0)
        p = p + _dot(oh, y[:, 64 * k:64 * (k + 1)])

    @pl.when(i == 0)
    def _():
        gsum_ref[...] = p

    @pl.when(i > 0)
    def _():
        gsum_ref[...] += p

    @pl.when(i == NBLK - 1)
    def _():
        pooled = gsum_ref[...] / jnp.maximum(gcnt_ref[...], 1.0)
        out_ref[...] = _dot(pooled, wout_ref[...]) + bout_ref[...]


def _pool_head_folded(hf, scf, shf, batf, gcnt2d, wout, bout2d):
    return pl.pallas_call(
        _pool_head_folded_body,
        grid=(NBLK,),
        in_specs=[pl.BlockSpec((RBF, 512), lambda i: (i, 0)),
                  pl.BlockSpec((1, 512), lambda i: (0, 0)),
                  pl.BlockSpec((1, 512), lambda i: (0, 0)),
                  pl.BlockSpec((1, 8, RBF), lambda i: (i, 0, 0)),
                  pl.BlockSpec((G, 1), lambda i: (0, 0)),
                  pl.BlockSpec((64, 2), lambda i: (0, 0)),
                  pl.BlockSpec((1, 2), lambda i: (0, 0))],
        out_specs=pl.BlockSpec((G, 2), lambda i: (0, 0)),
        out_shape=jax.ShapeDtypeStruct((G, 2), jnp.float32),
        scratch_shapes=[pltpu.VMEM((G, 64), jnp.float32)],
    )(hf, scf, shf, batf, gcnt2d, wout, bout2d)


def _bn_scale_shift_host(sums512, g, b):
    s = sums512.reshape(2, 8, 64).sum(axis=1)
    m = s[0] / float(N)
    v = s[1] / float(N) - m * m
    sc = g / jnp.sqrt(v + EPS)
    sh = b - m * sc
    return jnp.tile(sc, 8).reshape(1, 512), jnp.tile(sh, 8).reshape(1, 512)


def kernel(shape_id, color_id, pos_id, edge_index, batch, shape_emb,
           color_emb, pos_emb, W1l, b1l, W1r, g1, be1, W2l, b2l, W2r, g2,
           be2, Wout, bout):
    i32 = jnp.int32
    f32 = jnp.float32

    src = edge_index[0].astype(i32)
    dst = edge_index[1].astype(i32)
    srcr = jnp.concatenate([src, jnp.zeros((EPAD - E,), i32)])
    dstr = jnp.concatenate([dst, jnp.full((EPAD - E,), N, i32)])
    pad_n = jnp.zeros((NPAD - N,), i32)
    sid = jnp.concatenate([shape_id.astype(i32), pad_n])
    cid = jnp.concatenate([color_id.astype(i32), pad_n])
    pid = jnp.concatenate([pos_id.astype(i32), pad_n])
    batr = jnp.concatenate(
        [batch.astype(i32), jnp.full((NPAD - N,), 520, i32)])

    z16 = jnp.zeros((NT, 16), f32)
    z8 = jnp.zeros((NT, 8), f32)
    z32 = jnp.zeros((NT, 32), f32)
    z1 = jnp.zeros((NT,), f32)
    ones_h = jnp.ones((CHN,), f32)

    xs, xc, xp, aggs, aggc, aggp, cnt, gcnt = _sc1(
        sid, cid, pid, srcr, dstr, batr, shape_emb, color_emb, pos_emb,
        z16, z8, z1, ones_h)

    rcf = jnp.broadcast_to(
        (1.0 / jnp.maximum(cnt, 1.0)).reshape(NPAD, 1), (NPAD, 16)
    ).reshape(NF, 128)
    h1f, sums1 = _sage_dense_folded(
        [xs.reshape(NF, 128), xc.reshape(NF, 128)], [xp.reshape(NF, 64)],
        [aggs.reshape(NF, 128), aggc.reshape(NF, 128)], [aggp.reshape(NF, 64)],
        rcf, jnp.concatenate([W1r, W1l], axis=0), b1l.reshape(1, 64))

    sc1v, sh1v = _bn_scale_shift_host(sums1, g1, be1)
    ya, yb = _bn_relu_split_folded(h1f, sc1v, sh1v)

    srcp = 25600 * ((srcr % 8) // 4) + 4 * (srcr // 8) + (srcr % 4)
    agg2a, agg2b = _sc2(srcp, dstr, ya.reshape(NPAD, 32),
                        yb.reshape(NPAD, 32), z32)

    h2f, sums2 = _sage2_folded(
        ya, yb, agg2a.reshape(NF, 256), agg2b.reshape(NF, 256),
        rcf, jnp.concatenate([W2r, W2l], axis=0), b2l.reshape(1, 64))

    sc2v, sh2v = _bn_scale_shift_host(sums2, g2, be2)
    batf = batr.reshape(NBLK, RBF, 8).transpose(0, 2, 1)
    out = _pool_head_folded(
        h2f, sc2v, sh2v, batf,
        gcnt[:G].reshape(G, 1), Wout, bout.reshape(1, 2))

    return out


# speedup vs baseline: 1.1742x; 1.0031x over previous
import functools

import jax
import jax.numpy as jnp
from jax import lax
from jax.experimental import pallas as pl
from jax.experimental.pallas import tpu as pltpu
import jax.experimental.pallas.tpu_sc as plsc

N = 50000
E = 800000
G = 512
EPS = 1e-5

NPAD = 51200
EPAD = 819200
GP = 640
NC = 2
NS = 16
NT = NPAD // NS
ET = EPAD // NS
CHE1 = 512
NCH1 = ET // CHE1
CHE2 = 256
NB2 = 2
NCH2 = ET // CHE2
CHN = 640
NPCH = NT // CHN
RB = 2048
NBLK = NPAD // RB


def _mesh():
    return plsc.VectorSubcoreMesh(
        core_axis_name="c", subcore_axis_name="s", num_cores=NC, num_subcores=NS)


def _sc1_body(sid, cid, pid, srcr, dstr, batr, semb, cemb, pemb, z16, z8, z1,
              ones_h,
              xs, xc, xp, aggs, aggc, aggp, cnt, gcnt,
              acc16, acc8, cnt_sp, gcnt_sp,
              ei0, ei1, ed0, ed1, er16_0, er16_1, er8_0, er8_1,
              pi_v, pd_v, pr16, pr8, ones_e, ones_p,
              sem0, sem1, semp):
    c = lax.axis_index("c")
    s = lax.axis_index("s")
    nb = s * NT
    eb = s * ET
    ei = (ei0, ei1)
    ed = (ed0, ed1)
    er16 = (er16_0, er16_1)
    er8 = (er8_0, er8_1)
    sems = (sem0, sem1)
    pltpu.sync_copy(ones_h.at[pl.ds(0, CHE1)], ones_e)
    pltpu.sync_copy(ones_h, ones_p)

    pltpu.sync_copy(z16, acc16.at[pl.ds(nb, NT)])

    @pl.when(c == 0)
    def _():
        pltpu.sync_copy(z8, acc8.at[pl.ds(nb, NT)])

    @pl.when(c == 1)
    def _():
        pltpu.sync_copy(z1, cnt_sp.at[pl.ds(nb, NT)])

        @pl.when(s == 0)
        def _():
            pltpu.sync_copy(z1.at[pl.ds(0, GP)], gcnt_sp)

    plsc.subcore_barrier()

    @pl.when(c == 0)
    def _():
        @pl.loop(0, NPCH)
        def _(k):
            o = nb + k * CHN
            pltpu.sync_copy(sid.at[pl.ds(o, CHN)], pi_v)
            pltpu.sync_copy(pid.at[pl.ds(o, CHN)], pd_v)
            pltpu.async_copy(semb.at[pi_v], pr16, semp)
            pltpu.async_copy(pemb.at[pd_v], pr8, semp)
            pltpu.make_async_copy(xs.at[pl.ds(0, CHN)], pr16, semp).wait()
            pltpu.make_async_copy(xp.at[pl.ds(0, CHN)], pr8, semp).wait()
            pltpu.sync_copy(pr16, xs.at[pl.ds(o, CHN)])
            pltpu.sync_copy(pr8, xp.at[pl.ds(o, CHN)])

    @pl.when(c == 1)
    def _():
        @pl.loop(0, NPCH)
        def _(k):
            o = nb + k * CHN
            pltpu.sync_copy(cid.at[pl.ds(o, CHN)], pi_v)
            pltpu.sync_copy(batr.at[pl.ds(o, CHN)], pd_v)
            pltpu.async_copy(cemb.at[pi_v], pr16, semp).wait()
            pltpu.sync_copy(pr16, xc.at[pl.ds(o, CHN)])
            pltpu.sync_copy(ones_p, gcnt_sp.at[pd_v], add=True)

    plsc.subcore_barrier()

    @pl.when(c == 0)
    def _():
        for b in range(2):
            pltpu.sync_copy(srcr.at[pl.ds(eb + b * CHE1, CHE1)], ei[b])
            pltpu.sync_copy(dstr.at[pl.ds(eb + b * CHE1, CHE1)], ed[b])
            pltpu.async_copy(xs.at[ei[b]], er16[b], sems[b])
            pltpu.async_copy(xp.at[ei[b]], er8[b], sems[b])

        @pl.loop(0, NCH1, step=2)
        def _(k):
            for b in range(2):
                pltpu.make_async_copy(
                    xs.at[pl.ds(0, CHE1)], er16[b], sems[b]).wait()
                pltpu.make_async_copy(
                    xp.at[pl.ds(0, CHE1)], er8[b], sems[b]).wait()
                pltpu.sync_copy(er16[b], acc16.at[ed[b]], add=True)
                pltpu.sync_copy(er8[b], acc8.at[ed[b]], add=True)

                @pl.when(k + (b + 2) < NCH1)
                def _():
                    o = eb + (k + (b + 2)) * CHE1
                    pltpu.sync_copy(srcr.at[pl.ds(o, CHE1)], ei[b])
                    pltpu.sync_copy(dstr.at[pl.ds(o, CHE1)], ed[b])
                    pltpu.async_copy(xs.at[ei[b]], er16[b], sems[b])
                    pltpu.async_copy(xp.at[ei[b]], er8[b], sems[b])

    @pl.when(c == 1)
    def _():
        for b in range(2):
            pltpu.sync_copy(srcr.at[pl.ds(eb + b * CHE1, CHE1)], ei[b])
            pltpu.sync_copy(dstr.at[pl.ds(eb + b * CHE1, CHE1)], ed[b])
            pltpu.async_copy(xc.at[ei[b]], er16[b], sems[b])

        @pl.loop(0, NCH1, step=2)
        def _(k):
            for b in range(2):
                pltpu.make_async_copy(
                    xc.at[pl.ds(0, CHE1)], er16[b], sems[b]).wait()
                pltpu.sync_copy(er16[b], acc16.at[ed[b]], add=True)
                pltpu.sync_copy(ones_e, cnt_sp.at[ed[b]], add=True)

                @pl.when(k + (b + 2) < NCH1)
                def _():
                    o = eb + (k + (b + 2)) * CHE1
                    pltpu.sync_copy(srcr.at[pl.ds(o, CHE1)], ei[b])
                    pltpu.sync_copy(dstr.at[pl.ds(o, CHE1)], ed[b])
                    pltpu.async_copy(xc.at[ei[b]], er16[b], sems[b])

    plsc.subcore_barrier()

    @pl.when(c == 0)
    def _():
        pltpu.sync_copy(acc16.at[pl.ds(nb, NT)], aggs.at[pl.ds(nb, NT)])
        pltpu.sync_copy(acc8.at[pl.ds(nb, NT)], aggp.at[pl.ds(nb, NT)])

    @pl.when(c == 1)
    def _():
        pltpu.sync_copy(acc16.at[pl.ds(nb, NT)], aggc.at[pl.ds(nb, NT)])
        pltpu.sync_copy(cnt_sp.at[pl.ds(nb, NT)], cnt.at[pl.ds(nb, NT)])

        @pl.when(s == 0)
        def _():
            pltpu.sync_copy(gcnt_sp, gcnt)


def _sc1(sid, cid, pid, srcr, dstr, batr, semb, cemb, pemb, z16, z8, z1,
         ones_h):
    f32 = jnp.float32
    i32 = jnp.int32
    out_type = [
        jax.ShapeDtypeStruct((NPAD, 16), f32),
        jax.ShapeDtypeStruct((NPAD, 16), f32),
        jax.ShapeDtypeStruct((NPAD, 8), f32),
        jax.ShapeDtypeStruct((NPAD, 16), f32),
        jax.ShapeDtypeStruct((NPAD, 16), f32),
        jax.ShapeDtypeStruct((NPAD, 8), f32),
        jax.ShapeDtypeStruct((NPAD,), f32),
        jax.ShapeDtypeStruct((GP,), f32),
    ]
    scratch = [
        pltpu.VMEM_SHARED((NPAD, 16), f32),
        pltpu.VMEM_SHARED((NPAD, 8), f32),
        pltpu.VMEM_SHARED((NPAD,), f32),
        pltpu.VMEM_SHARED((GP,), f32),
        pltpu.VMEM((CHE1,), i32),
        pltpu.VMEM((CHE1,), i32),
        pltpu.VMEM((CHE1,), i32),
        pltpu.VMEM((CHE1,), i32),
        pltpu.VMEM((CHE1, 16), f32),
        pltpu.VMEM((CHE1, 16), f32),
        pltpu.VMEM((CHE1, 8), f32),
        pltpu.VMEM((CHE1, 8), f32),
        pltpu.VMEM((CHN,), i32),
        pltpu.VMEM((CHN,), i32),
        pltpu.VMEM((CHN, 16), f32),
        pltpu.VMEM((CHN, 8), f32),
        pltpu.VMEM((CHE1,), f32),
        pltpu.VMEM((CHN,), f32),
        pltpu.SemaphoreType.DMA,
        pltpu.SemaphoreType.DMA,
        pltpu.SemaphoreType.DMA,
    ]
    return pl.kernel(_sc1_body, out_type=out_type, mesh=_mesh(),
                     scratch_types=scratch,
                     compiler_params=pltpu.CompilerParams(
                         use_tc_tiling_on_sc=False))(
        sid, cid, pid, srcr, dstr, batr, semb, cemb, pemb, z16, z8, z1,
        ones_h)


def _sc2_body(srcr, dstr, y1a, y1b, z32, agg2a, agg2b, acc32, *rest):
    c = lax.axis_index("c")
    s = lax.axis_index("s")
    nb = s * NT
    eb = s * ET
    ei = rest[0:NB2]
    ed = rest[NB2:2 * NB2]
    er32 = rest[2 * NB2:3 * NB2]
    sems = rest[3 * NB2:4 * NB2]

    pltpu.sync_copy(z32, acc32.at[pl.ds(nb, NT)])
    plsc.subcore_barrier()

    def edge_loop(table):
        for b in range(NB2):
            pltpu.sync_copy(srcr.at[pl.ds(eb + b * CHE2, CHE2)], ei[b])
            pltpu.sync_copy(dstr.at[pl.ds(eb + b * CHE2, CHE2)], ed[b])
            pltpu.async_copy(table.at[ei[b]], er32[b], sems[b])

        @pl.loop(0, NCH2, step=NB2)
        def _(k):
            for b in range(NB2):
                pltpu.make_async_copy(
                    table.at[pl.ds(0, CHE2)], er32[b], sems[b]).wait()
                pltpu.sync_copy(er32[b], acc32.at[ed[b]], add=True)

                @pl.when(k + (b + NB2) < NCH2)
                def _():
                    o = eb + (k + (b + NB2)) * CHE2
                    pltpu.sync_copy(srcr.at[pl.ds(o, CHE2)], ei[b])
                    pltpu.sync_copy(dstr.at[pl.ds(o, CHE2)], ed[b])
                    pltpu.async_copy(table.at[ei[b]], er32[b], sems[b])

    @pl.when(c == 0)
    def _():
        edge_loop(y1a)

    @pl.when(c == 1)
    def _():
        edge_loop(y1b)

    plsc.subcore_barrier()

    @pl.when(c == 0)
    def _():
        pltpu.sync_copy(acc32.at[pl.ds(nb, NT)], agg2a.at[pl.ds(nb, NT)])

    @pl.when(c == 1)
    def _():
        pltpu.sync_copy(acc32.at[pl.ds(nb, NT)], agg2b.at[pl.ds(nb, NT)])


def _sc2(srcr, dstr, y1a, y1b, z32):
    f32 = jnp.float32
    i32 = jnp.int32
    out_type = [
        jax.ShapeDtypeStruct((NPAD, 32), f32),
        jax.ShapeDtypeStruct((NPAD, 32), f32),
    ]
    scratch = (
        [pltpu.VMEM_SHARED((NPAD, 32), f32)]
        + [pltpu.VMEM((CHE2,), i32)] * (2 * NB2)
        + [pltpu.VMEM((CHE2, 32), f32)] * NB2
        + [pltpu.SemaphoreType.DMA] * NB2
    )
    return pl.kernel(_sc2_body, out_type=out_type, mesh=_mesh(),
                     scratch_types=scratch,
                     compiler_params=pltpu.CompilerParams(
                         use_tc_tiling_on_sc=False))(srcr, dstr, y1a, y1b, z32)


def _dot(a, b):
    return jnp.dot(a, b, preferred_element_type=jnp.float32)


NF = NPAD // 8
RBF = NF // NBLK
NRF = N // 8


def _sage_dense_folded_body(x16w, xw, a16w, aw, rc_ref, w_ref, bl_ref,
                            h_ref, sums_ref):
    i = pl.program_id(0)
    cats = []
    for k in range(8):
        rc = rc_ref[:, 16 * k:16 * k + 1]
        xp = [r[:, (r.shape[1] // 8) * k:(r.shape[1] // 8) * (k + 1)]
              for r in x16w + xw]
        ap = [r[:, (r.shape[1] // 8) * k:(r.shape[1] // 8) * (k + 1)]
              for r in a16w + aw]
        cats.append(jnp.concatenate(xp + [a * rc for a in ap], axis=1))
    big = _dot(jnp.concatenate(cats, axis=0), w_ref[...]) + bl_ref[...]
    h = jnp.concatenate(
        [big[RBF * k:RBF * (k + 1)] for k in range(8)], axis=1)
    h_ref[...] = h
    rid = i * RBF + lax.broadcasted_iota(jnp.int32, (RBF, 1), 0)
    hm = jnp.where(rid < NRF, h, 0.0)
    ssum = jnp.concatenate(
        [jnp.sum(hm, axis=0, keepdims=True),
         jnp.sum(hm * hm, axis=0, keepdims=True)], axis=0)

    @pl.when(i == 0)
    def _():
        sums_ref[...] = ssum

    @pl.when(i > 0)
    def _():
        sums_ref[...] += ssum


def _sage_dense_folded(x16, xodd, a16, aodd, rcf, wcat, blf):
    n16 = len(x16)
    nod = len(xodd)

    def body(*refs):
        p = 0
        x16r = list(refs[p:p + n16]); p += n16
        xor_ = list(refs[p:p + nod]); p += nod
        a16r = list(refs[p:p + n16]); p += n16
        aor = list(refs[p:p + nod]); p += nod
        rc_ref, w_ref, bl_ref = refs[p], refs[p + 1], refs[p + 2]
        h_ref, sums_ref = refs[p + 3], refs[p + 4]
        _sage_dense_folded_body(x16r, xor_, a16r, aor, rc_ref, w_ref, bl_ref,
                                h_ref, sums_ref)

    row_spec = lambda a: pl.BlockSpec((RBF, a.shape[1]), lambda i: (i, 0))
    full_spec = lambda a: pl.BlockSpec(a.shape, lambda i: (0, 0))
    arrs = x16 + xodd + a16 + aodd
    in_specs = ([row_spec(a) for a in arrs]
                + [row_spec(rcf), full_spec(wcat), full_spec(blf)])
    return pl.pallas_call(
        body,
        grid=(NBLK,),
        in_specs=in_specs,
        out_specs=[pl.BlockSpec((RBF, 512), lambda i: (i, 0)),
                   pl.BlockSpec((2, 512), lambda i: (0, 0))],
        out_shape=[jax.ShapeDtypeStruct((NF, 512), jnp.float32),
                   jax.ShapeDtypeStruct((2, 512), jnp.float32)],
    )(*arrs, rcf, wcat, blf)


def _bn_relu_split_folded_body(h_ref, sc_ref, sh_ref, ya_ref, yb_ref):
    y = jnp.maximum(h_ref[...] * sc_ref[...] + sh_ref[...], 0.0)
    for j in range(2):
        ya_ref[j] = jnp.concatenate(
            [y[:, (4 * j + m) * 64:(4 * j + m) * 64 + 32] for m in range(4)],
            axis=1)
        yb_ref[j] = jnp.concatenate(
            [y[:, (4 * j + m) * 64 + 32:(4 * j + m + 1) * 64]
             for m in range(4)], axis=1)


def _bn_relu_split_folded(hf, scf, shf):
    return pl.pallas_call(
        _bn_relu_split_folded_body,
        grid=(NBLK,),
        in_specs=[pl.BlockSpec((RBF, 512), lambda i: (i, 0)),
                  pl.BlockSpec((1, 512), lambda i: (0, 0)),
                  pl.BlockSpec((1, 512), lambda i: (0, 0))],
        out_specs=[pl.BlockSpec((2, RBF, 128), lambda i: (0, i, 0)),
                   pl.BlockSpec((2, RBF, 128), lambda i: (0, i, 0))],
        out_shape=[jax.ShapeDtypeStruct((2, NF, 128), jnp.float32),
                   jax.ShapeDtypeStruct((2, NF, 128), jnp.float32)],
    )(hf, scf, shf)


def _sage2_folded_body(ya_ref, yb_ref, a2a_ref, a2b_ref, rc_ref, w_ref,
                       bl_ref, h_ref, sums_ref):
    i = pl.program_id(0)
    cats = []
    for k in range(8):
        j, m = divmod(k, 4)
        rc = rc_ref[:, 16 * k:16 * k + 1]
        y_k = jnp.concatenate(
            [ya_ref[j][:, 32 * m:32 * m + 32],
             yb_ref[j][:, 32 * m:32 * m + 32]], axis=1)
        a_k = jnp.concatenate(
            [a2a_ref[:, 32 * k:32 * k + 32],
             a2b_ref[:, 32 * k:32 * k + 32]], axis=1) * rc
        cats.append(jnp.concatenate([y_k, a_k], axis=1))
    big = _dot(jnp.concatenate(cats, axis=0), w_ref[...]) + bl_ref[...]
    h = jnp.concatenate(
        [big[RBF * k:RBF * (k + 1)] for k in range(8)], axis=1)
    h_ref[...] = h
    rid = i * RBF + lax.broadcasted_iota(jnp.int32, (RBF, 1), 0)
    hm = jnp.where(rid < NRF, h, 0.0)
    ssum = jnp.concatenate(
        [jnp.sum(hm, axis=0, keepdims=True),
         jnp.sum(hm * hm, axis=0, keepdims=True)], axis=0)

    @pl.when(i == 0)
    def _():
        sums_ref[...] = ssum

    @pl.when(i > 0)
    def _():
        sums_ref[...] += ssum


def _sage2_folded(ya, yb, a2af, a2bf, rcf, wcat, blf):
    full_spec = lambda a: pl.BlockSpec(a.shape, lambda i: (0, 0))
    return pl.pallas_call(
        _sage2_folded_body,
        grid=(NBLK,),
        in_specs=[pl.BlockSpec((2, RBF, 128), lambda i: (0, i, 0)),
                  pl.BlockSpec((2, RBF, 128), lambda i: (0, i, 0)),
                  pl.BlockSpec((RBF, 256), lambda i: (i, 0)),
                  pl.BlockSpec((RBF, 256), lambda i: (i, 0)),
                  pl.BlockSpec((RBF, 128), lambda i: (i, 0)),
                  full_spec(wcat), full_spec(blf)],
        out_specs=[pl.BlockSpec((RBF, 512), lambda i: (i, 0)),
                   pl.BlockSpec((2, 512), lambda i: (0, 0))],
        out_shape=[jax.ShapeDtypeStruct((NF, 512), jnp.float32),
                   jax.ShapeDtypeStruct((2, 512), jnp.float32)],
    )(ya, yb, a2af, a2bf, rcf, wcat, blf)


def _pool_head_folded_body(h_ref, sc_ref, sh_ref, bat_ref, gcnt_ref,
                           wout_ref, bout_ref, out_ref, gsum_ref):
    i = pl.program_id(0)
    y = jnp.maximum(h_ref[...] * sc_ref[...] + sh_ref[...], 0.0)
    p = jnp.zeros((G, 64), jnp.float32)
    gid = lax.broadcasted_iota(jnp.int32, (G, RBF), 0)
    for k in range(8):
        seg = bat_ref[0, k:k + 1, :]
        oh = jnp.where(gid == seg, 1.0, 0.0)
        p = p + _dot(oh, y[:, 64 * k:64 * (k + 1)])

    @pl.when(i == 0)
    def _():
        gsum_ref[...] = p

    @pl.when(i > 0)
    def _():
        gsum_ref[...] += p

    @pl.when(i == NBLK - 1)
    def _():
        pooled = gsum_ref[...] / jnp.maximum(gcnt_ref[...], 1.0)
        out_ref[...] = _dot(pooled, wout_ref[...]) + bout_ref[...]


def _pool_head_folded(hf, scf, shf, batf, gcnt2d, wout, bout2d):
    return pl.pallas_call(
        _pool_head_folded_body,
        grid=(NBLK,),
        in_specs=[pl.BlockSpec((RBF, 512), lambda i: (i, 0)),
                  pl.BlockSpec((1, 512), lambda i: (0, 0)),
                  pl.BlockSpec((1, 512), lambda i: (0, 0)),
                  pl.BlockSpec((1, 8, RBF), lambda i: (i, 0, 0)),
                  pl.BlockSpec((G, 1), lambda i: (0, 0)),
                  pl.BlockSpec((64, 2), lambda i: (0, 0)),
                  pl.BlockSpec((1, 2), lambda i: (0, 0))],
        out_specs=pl.BlockSpec((G, 2), lambda i: (0, 0)),
        out_shape=jax.ShapeDtypeStruct((G, 2), jnp.float32),
        scratch_shapes=[pltpu.VMEM((G, 64), jnp.float32)],
    )(hf, scf, shf, batf, gcnt2d, wout, bout2d)


def _bn_scale_shift_host(sums512, g, b):
    s = sums512.reshape(2, 8, 64).sum(axis=1)
    m = s[0] / float(N)
    v = s[1] / float(N) - m * m
    sc = g / jnp.sqrt(v + EPS)
    sh = b - m * sc
    return jnp.tile(sc, 8).reshape(1, 512), jnp.tile(sh, 8).reshape(1, 512)


def kernel(shape_id, color_id, pos_id, edge_index, batch, shape_emb,
           color_emb, pos_emb, W1l, b1l, W1r, g1, be1, W2l, b2l, W2r, g2,
           be2, Wout, bout):
    i32 = jnp.int32
    f32 = jnp.float32

    src = edge_index[0].astype(i32)
    dst = edge_index[1].astype(i32)
    srcr = jnp.concatenate([src, jnp.zeros((EPAD - E,), i32)])
    dstr = jnp.concatenate([dst, jnp.full((EPAD - E,), N, i32)])
    pad_n = jnp.zeros((NPAD - N,), i32)
    sid = jnp.concatenate([shape_id.astype(i32), pad_n])
    cid = jnp.concatenate([color_id.astype(i32), pad_n])
    pid = jnp.concatenate([pos_id.astype(i32), pad_n])
    batr = jnp.concatenate(
        [batch.astype(i32), jnp.full((NPAD - N,), 520, i32)])

    z16 = jnp.zeros((NT, 16), f32)
    z8 = jnp.zeros((NT, 8), f32)
    z32 = jnp.zeros((NT, 32), f32)
    z1 = jnp.zeros((NT,), f32)
    ones_h = jnp.ones((CHN,), f32)

    xs, xc, xp, aggs, aggc, aggp, cnt, gcnt = _sc1(
        sid, cid, pid, srcr, dstr, batr, shape_emb, color_emb, pos_emb,
        z16, z8, z1, ones_h)

    rcf = jnp.broadcast_to(
        (1.0 / jnp.maximum(cnt, 1.0)).reshape(NPAD, 1), (NPAD, 16)
    ).reshape(NF, 128)
    h1f, sums1 = _sage_dense_folded(
        [xs.reshape(NF, 128), xc.reshape(NF, 128)], [xp.reshape(NF, 64)],
        [aggs.reshape(NF, 128), aggc.reshape(NF, 128)], [aggp.reshape(NF, 64)],
        rcf, jnp.concatenate([W1r, W1l], axis=0), b1l.reshape(1, 64))

    sc1v, sh1v = _bn_scale_shift_host(sums1, g1, be1)
    ya, yb = _bn_relu_split_folded(h1f, sc1v, sh1v)

    srcp = 25600 * ((srcr % 8) // 4) + 4 * (srcr // 8) + (srcr % 4)
    agg2a, agg2b = _sc2(srcp, dstr, ya.reshape(NPAD, 32),
                        yb.reshape(NPAD, 32), z32)

    h2f, sums2 = _sage2_folded(
        ya, yb, agg2a.reshape(NF, 256), agg2b.reshape(NF, 256),
        rcf, jnp.concatenate([W2r, W2l], axis=0), b2l.reshape(1, 64))

    sc2v, sh2v = _bn_scale_shift_host(sums2, g2, be2)
    batf = batr.reshape(NBLK, RBF, 8).transpose(0, 2, 1)
    out = _pool_head_folded(
        h2f, sc2v, sh2v, batf,
        gcnt[:G].reshape(G, 1), Wout, bout.reshape(1, 2))

    return out
